# baseline (speedup 1.0000x reference)
import functools

import jax
import jax.numpy as jnp
from jax import lax
from jax.experimental import pallas as pl
from jax.experimental.pallas import tpu as pltpu
from jax.experimental.pallas import tpu_sc as plsc

B = 4096
T = 50
F = 128
D = 16
V = 1000000

_BB = 4096
_RB = 32


def _sc_gather_t(idx, tableT):
    info = plsc.get_sparse_core_info()
    nc, ns = info.num_cores, info.num_subcores
    nw = nc * ns
    b_per_w = B // nw
    mesh = plsc.VectorSubcoreMesh(core_axis_name="c", subcore_axis_name="s")

    @functools.partial(
        pl.kernel,
        mesh=mesh,
        out_type=jax.ShapeDtypeStruct((2, D // 2, B), jnp.float32),
        scratch_types=[
            pltpu.VMEM((b_per_w,), jnp.int32),
            pltpu.VMEM((b_per_w,), jnp.int32),
            pltpu.VMEM((_RB, 2, D // 2, 128), jnp.float32),
            pltpu.VMEM((2, D // 2, b_per_w), jnp.float32),
            pltpu.SemaphoreType.DMA,
        ],
        compiler_params=pltpu.CompilerParams(
            needs_layout_passes=False, use_tc_tiling_on_sc=True
        ),
    )
    def gather_kernel(idx_hbm, tab_hbm, out_hbm, idx_v, off_v, stage, vals, sem):
        wid = lax.axis_index("s") * nc + lax.axis_index("c")
        base = wid * b_per_w
        pltpu.sync_copy(idx_hbm.at[pl.ds(base, b_per_w)], idx_v)
        lane = lax.iota(jnp.int32, 16)
        for r in range(b_per_w // _RB):
            copies = []
            for g in range(_RB // 16):
                chunk = idx_v[pl.ds(r * _RB + g * 16, 16)]
                off_v[pl.ds(g * 16, 16)] = jnp.bitwise_and(chunk, 127)
                tiles = jnp.bitwise_and(chunk, ~127)
                for j in range(16):
                    v0 = pl.multiple_of(
                        jnp.sum(jnp.where(lane == j, tiles, 0)), 128
                    )
                    copies.append(
                        pltpu.async_copy(
                            tab_hbm.at[:, :, pl.ds(v0, 128)],
                            stage.at[g * 16 + j],
                            sem,
                        )
                    )
            for c in copies:
                c.wait()
            for g in range(_RB // 16):
                i0 = lane + g * 16
                voff = off_v[pl.ds(g * 16, 16)]
                for tr in range(2):
                    for d in range(D // 2):
                        col = plsc.load_gather(
                            stage,
                            [i0, jnp.full((16,), tr, jnp.int32),
                             jnp.full((16,), d, jnp.int32), voff],
                        )
                        vals[tr, d, pl.ds(r * _RB + g * 16, 16)] = col
        pltpu.sync_copy(vals, out_hbm.at[:, :, pl.ds(base, b_per_w)])

    return gather_kernel(idx, tableT)


def _concat_t_body(x_ref, e_ref, o_ref):
    o_ref[0, 0:F, :] = jnp.transpose(x_ref[0], (1, 0))
    o_ref[0, F : F + D, :] = e_ref[...]


def _tc_concat_t(xT, eT):
    grid = (T, B // _BB)
    return pl.pallas_call(
        _concat_t_body,
        grid=grid,
        in_specs=[
            pl.BlockSpec((1, _BB, F), lambda t, j: (t, j, 0)),
            pl.BlockSpec((D, _BB), lambda t, j: (0, j)),
        ],
        out_specs=pl.BlockSpec((1, F + D, _BB), lambda t, j: (t, 0, j)),
        out_shape=jax.ShapeDtypeStruct((T, F + D, B), jnp.float32),
    )(xT, eT)


def kernel(x, ticker, emb_table):
    xT = jnp.transpose(x, (1, 0, 2))
    tab3 = jnp.reshape(jnp.transpose(emb_table, (1, 0)), (2, 8, V))
    idx = jnp.reshape(ticker, (B,)).astype(jnp.int32)
    eT = jnp.reshape(_sc_gather_t(idx, tab3), (D, B))
    outT = _tc_concat_t(xT, eT)
    return jnp.transpose(outT, (2, 0, 1))
